# Initial kernel scaffold; baseline (speedup 1.0000x reference)
import jax
import jax.numpy as jnp
from jax.experimental import pallas as pl


def kernel(x, topk_indices, topk_weights):
    raise NotImplementedError("write your pallas kernel here")


# speedup vs baseline: 13.3758x; 13.3758x over previous
import jax
import jax.numpy as jnp
from jax.experimental import pallas as pl


_BLOCK_T = 512


def _combine_kernel(x_ref, w_ref, o_ref):
    wsum = jnp.sum(w_ref[...], axis=1, keepdims=True)
    o_ref[...] = x_ref[...] * wsum


def kernel(x, topk_indices, topk_weights):
    del topk_indices
    t, h = x.shape
    k = topk_weights.shape[1]
    grid = (t // _BLOCK_T,)
    return pl.pallas_call(
        _combine_kernel,
        grid=grid,
        in_specs=[
            pl.BlockSpec((_BLOCK_T, h), lambda i: (i, 0)),
            pl.BlockSpec((_BLOCK_T, k), lambda i: (i, 0)),
        ],
        out_specs=pl.BlockSpec((_BLOCK_T, h), lambda i: (i, 0)),
        out_shape=jax.ShapeDtypeStruct((t, h), x.dtype),
    )(x, topk_weights)
